# baseline (speedup 1.0000x reference)
import functools

import jax
import jax.numpy as jnp
from jax import lax
from jax.experimental import pallas as pl
from jax.experimental.pallas import tpu as pltpu, tpu_sc as plsc

NC = 2
NS = 16
NW = NC * NS


def _sc_mesh():
    return plsc.VectorSubcoreMesh(core_axis_name="c", subcore_axis_name="s")


def _deg_kernel(E, N_pad, CH, K):
    EPW = E // NW
    NCH = EPW // CH
    NBLK = NCH // K
    RPW = N_pad // NS

    @functools.partial(
        pl.kernel,
        out_type=jax.ShapeDtypeStruct((NC, N_pad), jnp.float32),
        mesh=_sc_mesh(),
        scratch_types=[
            pltpu.VMEM((NCH, CH), jnp.int32),
            pltpu.VMEM((128,), jnp.float32),
            pltpu.VMEM_SHARED((N_pad,), jnp.float32),
            pltpu.SemaphoreType.DMA((K,)),
        ],
        compiler_params=pltpu.CompilerParams(use_tc_tiling_on_sc=False),
    )
    def deg_k(dst_hbm, zeros_hbm, out_hbm, idx_v, ones_v, acc_sh, ssem):
        c = lax.axis_index("c")
        s = lax.axis_index("s")
        wid = s * NC + c
        for j in range(8):
            ones_v[pl.ds(j * 16, 16)] = jnp.ones((16,), jnp.float32)
        pltpu.sync_copy(dst_hbm.at[wid], idx_v)
        pltpu.sync_copy(zeros_hbm.at[pl.ds(s * RPW, RPW)],
                        acc_sh.at[pl.ds(s * RPW, RPW)])
        plsc.subcore_barrier()

        def block(gb, carry):
            base = gb * K
            sds = [pltpu.async_copy(ones_v.at[pl.ds(0, CH)],
                                    acc_sh.at[idx_v.at[base + k]],
                                    ssem.at[k], add=True)
                   for k in range(K)]
            for k in range(K):
                sds[k].wait()
            return carry

        lax.fori_loop(0, NBLK, block, 0)
        plsc.subcore_barrier()
        pltpu.sync_copy(acc_sh.at[pl.ds(s * RPW, RPW)],
                        out_hbm.at[c, pl.ds(s * RPW, RPW)])

    return deg_k


def _agg_kernel(E, N, N_pad, D, CH, K):
    EPW = E // NW
    NCH = EPW // CH
    NBLK = NCH // K
    RPW = N_pad // NS

    @functools.partial(
        pl.kernel,
        out_type=jax.ShapeDtypeStruct((NC, N_pad, D), jnp.bfloat16),
        mesh=_sc_mesh(),
        scratch_types=[
            pltpu.VMEM((NCH, CH), jnp.int32),
            pltpu.VMEM((NCH, CH), jnp.int32),
        ] + [pltpu.VMEM((CH, D), jnp.bfloat16) for _ in range(2 * K)] + [
            pltpu.VMEM_SHARED((N_pad, D), jnp.bfloat16),
            pltpu.SemaphoreType.DMA((K,)),
            pltpu.SemaphoreType.DMA((K,)),
            pltpu.SemaphoreType.DMA((K,)),
            pltpu.SemaphoreType.DMA((K,)),
        ],
        compiler_params=pltpu.CompilerParams(use_tc_tiling_on_sc=False),
    )
    def agg_k(src_hbm, dst_hbm, ht_hbm, zeros_hbm, out_hbm,
              isrc_v, idst_v, *rest):
        rows = rest[:2 * K]
        acc_sh = rest[2 * K]
        gsem_a, gsem_b, ssem_a, ssem_b = rest[2 * K + 1:2 * K + 5]
        c = lax.axis_index("c")
        s = lax.axis_index("s")
        wid = s * NC + c
        pltpu.sync_copy(src_hbm.at[wid], isrc_v)
        pltpu.sync_copy(dst_hbm.at[wid], idst_v)
        pltpu.sync_copy(zeros_hbm.at[pl.ds(s * RPW, RPW)],
                        acc_sh.at[pl.ds(s * RPW, RPW)])
        plsc.subcore_barrier()

        rows_a, rows_b = rows[:K], rows[K:]

        def fire_gathers(blk, rset, sem):
            base = blk * K
            return [pltpu.async_copy(ht_hbm.at[isrc_v.at[base + k]],
                                     rset[k], sem.at[k])
                    for k in range(K)]

        def fire_scatters(blk, rset, sem):
            base = blk * K
            return [pltpu.async_copy(rset[k], acc_sh.at[idst_v.at[base + k]],
                                     sem.at[k], add=True)
                    for k in range(K)]

        def body(g2, carry):
            b0 = g2 * 2
            gA = fire_gathers(b0, rows_a, gsem_a)
            gB = fire_gathers(b0 + 1, rows_b, gsem_b)
            for d in gA:
                d.wait()
            sA = fire_scatters(b0, rows_a, ssem_a)
            for d in gB:
                d.wait()
            sB = fire_scatters(b0 + 1, rows_b, ssem_b)
            for d in sA:
                d.wait()
            for d in sB:
                d.wait()
            return carry

        lax.fori_loop(0, NBLK // 2, body, 0)
        plsc.subcore_barrier()
        pltpu.sync_copy(acc_sh.at[pl.ds(s * RPW, RPW)],
                        out_hbm.at[c, pl.ds(s * RPW, RPW)])

    return agg_k


def _tc1(n, x_ref, w_ref, degp_ref, ht_ref, dis_ref):
    deg = degp_ref[0, :n, :] + degp_ref[1, :n, :] + 1.0
    dis = lax.rsqrt(deg)
    h = jnp.dot(x_ref[...], w_ref[...], preferred_element_type=jnp.float32)
    ht_ref[...] = (h * dis).astype(jnp.bfloat16)
    dis_ref[...] = dis


def _tc2(n, aggp_ref, ht_ref, dis_ref, b_ref, w_ref, out_ref):
    agg = (aggp_ref[0, :n, :].astype(jnp.float32)
           + aggp_ref[1, :n, :].astype(jnp.float32))
    t = (agg + ht_ref[...].astype(jnp.float32)) * dis_ref[...] + b_ref[...]
    o = jnp.maximum(t, 0.0)
    out_ref[...] = (jnp.dot(o, w_ref[...], preferred_element_type=jnp.float32)
                    * dis_ref[...]).astype(jnp.bfloat16)


def _tc3(n, aggp_ref, ht_ref, dis_ref, b_ref, wfc_ref, bfc_ref, out_ref):
    agg = (aggp_ref[0, :n, :].astype(jnp.float32)
           + aggp_ref[1, :n, :].astype(jnp.float32))
    t = (agg + ht_ref[...].astype(jnp.float32)) * dis_ref[...] + b_ref[...]
    o = jnp.maximum(t, 0.0)
    m = jnp.sum(o, axis=0, keepdims=True) * (1.0 / n)
    out_ref[...] = jnp.dot(m, wfc_ref[...],
                           preferred_element_type=jnp.float32) + bfc_ref[...]


def kernel(x, edge_index, W1, b1, W2, b2, Wfc, bfc):
    N, D_in = x.shape
    E = edge_index.shape[1]
    D1 = W1.shape[1]
    D2 = W2.shape[1]
    C = Wfc.shape[1]
    N_pad = ((N + NW * 16 - 1) // (NW * 16)) * (NW * 16)
    CH = 125
    K1, K2 = 4, 5
    NCH = (E // NW) // CH
    assert E % (NW * CH) == 0 and NCH % (2 * K1) == 0 and NCH % (2 * K2) == 0

    er = edge_index.astype(jnp.int32).reshape(2, NW, NCH, CH)
    src_r = er[0]
    dst_r = er[1]
    zeros1 = jnp.zeros((N_pad,), jnp.float32)

    degp = _deg_kernel(E, N_pad, CH, 8)(dst_r, zeros1)[:, :, None]

    ht1, dis = pl.pallas_call(
        functools.partial(_tc1, N),
        out_shape=[
            jax.ShapeDtypeStruct((N, D1), jnp.bfloat16),
            jax.ShapeDtypeStruct((N, 1), jnp.float32),
        ],
    )(x, W1, degp)

    zeros2 = jnp.zeros((N_pad, D1), jnp.bfloat16)
    aggp1 = _agg_kernel(E, N, N_pad, D1, CH, K1)(src_r, dst_r, ht1, zeros2)

    ht2 = pl.pallas_call(
        functools.partial(_tc2, N),
        out_shape=jax.ShapeDtypeStruct((N, D2), jnp.bfloat16),
    )(aggp1, ht1, dis, b1, W2)

    zeros3 = jnp.zeros((N_pad, D2), jnp.bfloat16)
    aggp2 = _agg_kernel(E, N, N_pad, D2, CH, K2)(src_r, dst_r, ht2, zeros3)

    out = pl.pallas_call(
        functools.partial(_tc3, N),
        out_shape=jax.ShapeDtypeStruct((1, C), jnp.float32),
    )(aggp2, ht2, dis, b2, Wfc, bfc)
    return out.reshape(C)


# speedup vs baseline: 1.0963x; 1.0963x over previous
import functools

import jax
import jax.numpy as jnp
from jax import lax
from jax.experimental import pallas as pl
from jax.experimental.pallas import tpu as pltpu, tpu_sc as plsc

NC = 2
NS = 16
NW = NC * NS


def _sc_mesh():
    return plsc.VectorSubcoreMesh(core_axis_name="c", subcore_axis_name="s")


def _deg_kernel(E, N_pad, CH, K):
    EPW = E // NW
    NCH = EPW // CH
    NBLK = NCH // K
    RPW = N_pad // NS

    @functools.partial(
        pl.kernel,
        out_type=jax.ShapeDtypeStruct((NC, N_pad), jnp.float32),
        mesh=_sc_mesh(),
        scratch_types=[
            pltpu.VMEM((NCH, CH), jnp.int32),
            pltpu.VMEM((128,), jnp.float32),
            pltpu.VMEM_SHARED((N_pad,), jnp.float32),
            pltpu.SemaphoreType.DMA((K,)),
        ],
        compiler_params=pltpu.CompilerParams(use_tc_tiling_on_sc=False),
    )
    def deg_k(edges_hbm, zeros_hbm, out_hbm, idx_v, ones_v, acc_sh, ssem):
        c = lax.axis_index("c")
        s = lax.axis_index("s")
        wid = s * NC + c
        for j in range(8):
            ones_v[pl.ds(j * 16, 16)] = jnp.ones((16,), jnp.float32)
        pltpu.sync_copy(edges_hbm.at[1, wid], idx_v)
        pltpu.sync_copy(zeros_hbm.at[pl.ds(s * RPW, RPW)],
                        acc_sh.at[pl.ds(s * RPW, RPW)])
        plsc.subcore_barrier()

        def block(gb, carry):
            base = gb * K
            sds = [pltpu.async_copy(ones_v.at[pl.ds(0, CH)],
                                    acc_sh.at[idx_v.at[base + k]],
                                    ssem.at[k], add=True)
                   for k in range(K)]
            for k in range(K):
                sds[k].wait()
            return carry

        lax.fori_loop(0, NBLK, block, 0)
        plsc.subcore_barrier()
        pltpu.sync_copy(acc_sh.at[pl.ds(s * RPW, RPW)],
                        out_hbm.at[c, pl.ds(s * RPW, RPW)])

    return deg_k


def _agg_kernel(E, N, N_pad, D, CH, K):
    EPW = E // NW
    NCH = EPW // CH
    NBLK = NCH // K
    RPW = N_pad // NS

    @functools.partial(
        pl.kernel,
        out_type=jax.ShapeDtypeStruct((NC, N_pad, D), jnp.bfloat16),
        mesh=_sc_mesh(),
        scratch_types=[
            pltpu.VMEM((NCH, CH), jnp.int32),
            pltpu.VMEM((NCH, CH), jnp.int32),
        ] + [pltpu.VMEM((CH, D), jnp.bfloat16) for _ in range(2 * K)] + [
            pltpu.VMEM_SHARED((N_pad, D), jnp.bfloat16),
            pltpu.SemaphoreType.DMA((K,)),
            pltpu.SemaphoreType.DMA((K,)),
            pltpu.SemaphoreType.DMA((K,)),
            pltpu.SemaphoreType.DMA((K,)),
        ],
        compiler_params=pltpu.CompilerParams(use_tc_tiling_on_sc=False),
    )
    def agg_k(edges_hbm, ht_hbm, zeros_hbm, out_hbm,
              isrc_v, idst_v, *rest):
        rows = rest[:2 * K]
        acc_sh = rest[2 * K]
        gsem_a, gsem_b, ssem_a, ssem_b = rest[2 * K + 1:2 * K + 5]
        c = lax.axis_index("c")
        s = lax.axis_index("s")
        wid = s * NC + c
        pltpu.sync_copy(edges_hbm.at[0, wid], isrc_v)
        pltpu.sync_copy(edges_hbm.at[1, wid], idst_v)
        pltpu.sync_copy(zeros_hbm.at[pl.ds(s * RPW, RPW)],
                        acc_sh.at[pl.ds(s * RPW, RPW)])
        plsc.subcore_barrier()

        rows_a, rows_b = rows[:K], rows[K:]

        def fire_gathers(blk, rset, sem):
            base = blk * K
            return [pltpu.async_copy(ht_hbm.at[isrc_v.at[base + k]],
                                     rset[k], sem.at[k])
                    for k in range(K)]

        def fire_scatters(blk, rset, sem):
            base = blk * K
            return [pltpu.async_copy(rset[k], acc_sh.at[idst_v.at[base + k]],
                                     sem.at[k], add=True)
                    for k in range(K)]

        def body(g2, carry):
            b0 = g2 * 2
            gA = fire_gathers(b0, rows_a, gsem_a)
            gB = fire_gathers(b0 + 1, rows_b, gsem_b)
            for d in gA:
                d.wait()
            sA = fire_scatters(b0, rows_a, ssem_a)
            for d in gB:
                d.wait()
            sB = fire_scatters(b0 + 1, rows_b, ssem_b)
            for d in sA:
                d.wait()
            for d in sB:
                d.wait()
            return carry

        lax.fori_loop(0, NBLK // 2, body, 0)
        plsc.subcore_barrier()
        pltpu.sync_copy(acc_sh.at[pl.ds(s * RPW, RPW)],
                        out_hbm.at[c, pl.ds(s * RPW, RPW)])

    return agg_k


def _tc1(n, x_ref, w_ref, degp_ref, ht_ref, dis_ref):
    deg = degp_ref[0, :n, :] + degp_ref[1, :n, :] + 1.0
    dis = lax.rsqrt(deg)
    h = jnp.dot(x_ref[...], w_ref[...], preferred_element_type=jnp.float32)
    ht_ref[...] = (h * dis).astype(jnp.bfloat16)
    dis_ref[...] = dis


def _tc2(n, aggp_ref, ht_ref, dis_ref, b_ref, w_ref, out_ref):
    agg = (aggp_ref[0, :n, :].astype(jnp.float32)
           + aggp_ref[1, :n, :].astype(jnp.float32))
    t = (agg + ht_ref[...].astype(jnp.float32)) * dis_ref[...] + b_ref[...]
    o = jnp.maximum(t, 0.0)
    out_ref[...] = (jnp.dot(o, w_ref[...], preferred_element_type=jnp.float32)
                    * dis_ref[...]).astype(jnp.bfloat16)


def _tc3(n, aggp_ref, ht_ref, dis_ref, b_ref, wfc_ref, bfc_ref, out_ref):
    agg = (aggp_ref[0, :n, :].astype(jnp.float32)
           + aggp_ref[1, :n, :].astype(jnp.float32))
    t = (agg + ht_ref[...].astype(jnp.float32)) * dis_ref[...] + b_ref[...]
    o = jnp.maximum(t, 0.0)
    m = jnp.sum(o, axis=0, keepdims=True) * (1.0 / n)
    out_ref[...] = jnp.dot(m, wfc_ref[...],
                           preferred_element_type=jnp.float32) + bfc_ref[...]


def kernel(x, edge_index, W1, b1, W2, b2, Wfc, bfc):
    N, D_in = x.shape
    E = edge_index.shape[1]
    D1 = W1.shape[1]
    D2 = W2.shape[1]
    C = Wfc.shape[1]
    N_pad = ((N + NW * 16 - 1) // (NW * 16)) * (NW * 16)
    CH = 125
    K1, K2 = 4, 5
    NCH = (E // NW) // CH
    assert E % (NW * CH) == 0 and NCH % (2 * K1) == 0 and NCH % (2 * K2) == 0

    er = edge_index.astype(jnp.int32).reshape(2, NW, NCH, CH)
    zeros1 = jnp.zeros((N_pad,), jnp.float32)

    degp = _deg_kernel(E, N_pad, CH, 8)(er, zeros1)[:, :, None]

    ht1, dis = pl.pallas_call(
        functools.partial(_tc1, N),
        out_shape=[
            jax.ShapeDtypeStruct((N, D1), jnp.bfloat16),
            jax.ShapeDtypeStruct((N, 1), jnp.float32),
        ],
    )(x, W1, degp)

    zeros2 = jnp.zeros((N_pad, D1), jnp.bfloat16)
    aggp1 = _agg_kernel(E, N, N_pad, D1, CH, K1)(er, ht1, zeros2)

    ht2 = pl.pallas_call(
        functools.partial(_tc2, N),
        out_shape=jax.ShapeDtypeStruct((N, D2), jnp.bfloat16),
    )(aggp1, ht1, dis, b1, W2)

    zeros3 = jnp.zeros((N_pad, D2), jnp.bfloat16)
    aggp2 = _agg_kernel(E, N, N_pad, D2, CH, K2)(er, ht2, zeros3)

    out = pl.pallas_call(
        functools.partial(_tc3, N),
        out_shape=jax.ShapeDtypeStruct((1, C), jnp.float32),
    )(aggp2, ht2, dis, b2, Wfc, bfc)
    return out.reshape(C)


# speedup vs baseline: 1.1379x; 1.0379x over previous
import functools

import jax
import jax.numpy as jnp
from jax import lax
from jax.experimental import pallas as pl
from jax.experimental.pallas import tpu as pltpu, tpu_sc as plsc

NC = 2
NS = 16
NW = NC * NS


def _sc_mesh():
    return plsc.VectorSubcoreMesh(core_axis_name="c", subcore_axis_name="s")


def _deg_kernel(E, N_pad, CH, K):
    EPW = E // NW
    NCH = EPW // CH
    NBLK = NCH // K
    RPW = N_pad // NS

    @functools.partial(
        pl.kernel,
        out_type=jax.ShapeDtypeStruct((NC, N_pad), jnp.float32),
        mesh=_sc_mesh(),
        scratch_types=[
            pltpu.VMEM((NCH, CH), jnp.int32),
            pltpu.VMEM((128,), jnp.float32),
            pltpu.VMEM_SHARED((N_pad,), jnp.float32),
            pltpu.SemaphoreType.DMA((K,)),
        ],
        compiler_params=pltpu.CompilerParams(use_tc_tiling_on_sc=False),
    )
    def deg_k(edges_hbm, zeros_hbm, out_hbm, idx_v, ones_v, acc_sh, ssem):
        c = lax.axis_index("c")
        s = lax.axis_index("s")
        wid = s * NC + c
        for j in range(8):
            ones_v[pl.ds(j * 16, 16)] = jnp.ones((16,), jnp.float32)
        pltpu.sync_copy(edges_hbm.at[1, wid], idx_v)
        pltpu.sync_copy(zeros_hbm.at[pl.ds(s * RPW, RPW)],
                        acc_sh.at[pl.ds(s * RPW, RPW)])
        plsc.subcore_barrier()

        def block(gb, carry):
            base = gb * K
            sds = [pltpu.async_copy(ones_v.at[pl.ds(0, CH)],
                                    acc_sh.at[idx_v.at[base + k]],
                                    ssem.at[k], add=True)
                   for k in range(K)]
            for k in range(K):
                sds[k].wait()
            return carry

        lax.fori_loop(0, NBLK, block, 0)
        plsc.subcore_barrier()
        pltpu.sync_copy(acc_sh.at[pl.ds(s * RPW, RPW)],
                        out_hbm.at[c, pl.ds(s * RPW, RPW)])

    return deg_k


def _agg_kernel(E, N, N_pad, D, CH, K):
    EPW = E // NW
    NCH = EPW // CH
    NBLK = NCH // K
    RPW = N_pad // NS

    @functools.partial(
        pl.kernel,
        out_type=jax.ShapeDtypeStruct((NC, N_pad, D), jnp.bfloat16),
        mesh=_sc_mesh(),
        scratch_types=[
            pltpu.VMEM((NCH, CH), jnp.int32),
            pltpu.VMEM((NCH, CH), jnp.int32),
        ] + [pltpu.VMEM((CH, D), jnp.bfloat16) for _ in range(2 * K)] + [
            pltpu.VMEM_SHARED((N_pad, D), jnp.bfloat16),
            pltpu.SemaphoreType.DMA((K,)),
            pltpu.SemaphoreType.DMA((K,)),
            pltpu.SemaphoreType.DMA((K,)),
            pltpu.SemaphoreType.DMA((K,)),
        ],
        compiler_params=pltpu.CompilerParams(use_tc_tiling_on_sc=False),
    )
    def agg_k(edges_hbm, ht_hbm, zeros_hbm, out_hbm,
              isrc_v, idst_v, *rest):
        rows = rest[:2 * K]
        acc_sh = rest[2 * K]
        gsem_a, gsem_b, ssem_a, ssem_b = rest[2 * K + 1:2 * K + 5]
        c = lax.axis_index("c")
        s = lax.axis_index("s")
        wid = s * NC + c
        pltpu.sync_copy(edges_hbm.at[0, wid], isrc_v)
        pltpu.sync_copy(edges_hbm.at[1, wid], idst_v)
        pltpu.sync_copy(zeros_hbm.at[pl.ds(s * RPW, RPW)],
                        acc_sh.at[pl.ds(s * RPW, RPW)])
        plsc.subcore_barrier()

        rows_a, rows_b = rows[:K], rows[K:]

        def fire_gathers(blk, rset, sem):
            base = blk * K
            return [pltpu.async_copy(ht_hbm.at[isrc_v.at[base + k]],
                                     rset[k], sem.at[k])
                    for k in range(K)]

        def fire_scatters(blk, rset, sem):
            base = blk * K
            return [pltpu.async_copy(rset[k], acc_sh.at[idst_v.at[base + k]],
                                     sem.at[k], add=True)
                    for k in range(K)]

        def body(g2, carry):
            b0 = g2 * 2
            gA = fire_gathers(b0, rows_a, gsem_a)
            gB = fire_gathers(b0 + 1, rows_b, gsem_b)
            for d in gA:
                d.wait()
            sA = fire_scatters(b0, rows_a, ssem_a)
            for d in gB:
                d.wait()
            sB = fire_scatters(b0 + 1, rows_b, ssem_b)
            for d in sA:
                d.wait()
            for d in sB:
                d.wait()
            return carry

        lax.fori_loop(0, NBLK // 2, body, 0)
        plsc.subcore_barrier()
        pltpu.sync_copy(acc_sh.at[pl.ds(s * RPW, RPW)],
                        out_hbm.at[c, pl.ds(s * RPW, RPW)])

    return agg_k


def _tc1(n, x_ref, w_ref, d0_ref, d1_ref, ht_ref, dis_ref):
    deg = d0_ref[...] + d1_ref[...] + 1.0
    dis = lax.rsqrt(deg)
    h = jnp.dot(x_ref[...], w_ref[...], preferred_element_type=jnp.float32)
    ht_ref[...] = (h * dis).astype(jnp.bfloat16)
    dis_ref[...] = dis


def _tc2(n, aggp_ref, ht_ref, dis_ref, b_ref, w_ref, out_ref):
    agg = (aggp_ref[0, :n, :].astype(jnp.float32)
           + aggp_ref[1, :n, :].astype(jnp.float32))
    t = (agg + ht_ref[...].astype(jnp.float32)) * dis_ref[...] + b_ref[...]
    o = jnp.maximum(t, 0.0)
    out_ref[...] = (jnp.dot(o, w_ref[...], preferred_element_type=jnp.float32)
                    * dis_ref[...]).astype(jnp.bfloat16)


def _tc3(n, aggp_ref, ht_ref, dis_ref, b_ref, wfc_ref, bfc_ref, out_ref):
    agg = (aggp_ref[0, :n, :].astype(jnp.float32)
           + aggp_ref[1, :n, :].astype(jnp.float32))
    t = (agg + ht_ref[...].astype(jnp.float32)) * dis_ref[...] + b_ref[...]
    o = jnp.maximum(t, 0.0)
    m = jnp.sum(o, axis=0, keepdims=True) * (1.0 / n)
    out_ref[...] = jnp.dot(m, wfc_ref[...],
                           preferred_element_type=jnp.float32) + bfc_ref[...]


def kernel(x, edge_index, W1, b1, W2, b2, Wfc, bfc):
    N, D_in = x.shape
    E = edge_index.shape[1]
    D1 = W1.shape[1]
    D2 = W2.shape[1]
    C = Wfc.shape[1]
    N_pad = ((N + NW * 16 - 1) // (NW * 16)) * (NW * 16)
    CH = 125
    K1, K2 = 5, 5
    NCH = (E // NW) // CH
    assert E % (NW * CH) == 0 and NCH % (2 * K1) == 0 and NCH % (2 * K2) == 0

    er = edge_index.astype(jnp.int32).reshape(2, NW, NCH, CH)
    zeros1 = jnp.zeros((N_pad,), jnp.float32)

    degp = _deg_kernel(E, N_pad, CH, 8)(er, zeros1)
    d0 = degp[0, :N].reshape(N, 1)
    d1 = degp[1, :N].reshape(N, 1)

    ht1, dis = pl.pallas_call(
        functools.partial(_tc1, N),
        out_shape=[
            jax.ShapeDtypeStruct((N, D1), jnp.bfloat16),
            jax.ShapeDtypeStruct((N, 1), jnp.float32),
        ],
    )(x, W1, d0, d1)

    zeros2 = jnp.zeros((N_pad, D1), jnp.bfloat16)
    aggp1 = _agg_kernel(E, N, N_pad, D1, CH, K1)(er, ht1, zeros2)

    ht2 = pl.pallas_call(
        functools.partial(_tc2, N),
        out_shape=jax.ShapeDtypeStruct((N, D2), jnp.bfloat16),
    )(aggp1, ht1, dis, b1, W2)

    zeros3 = jnp.zeros((N_pad, D2), jnp.bfloat16)
    aggp2 = _agg_kernel(E, N, N_pad, D2, CH, K2)(er, ht2, zeros3)

    out = pl.pallas_call(
        functools.partial(_tc3, N),
        out_shape=jax.ShapeDtypeStruct((1, C), jnp.float32),
    )(aggp2, ht2, dis, b2, Wfc, bfc)
    return out.reshape(C)
